# speedup vs baseline: 13.3699x; 13.3699x over previous
import functools

import jax
import jax.numpy as jnp
from jax import lax
from jax.experimental import pallas as pl
from jax.experimental.pallas import tpu as pltpu
from jax.experimental.pallas import tpu_sc as plsc

NC = 2
NS = 16
L = 16
CH = 128


def _sc_mesh():
    return plsc.VectorSubcoreMesh(core_axis_name="c", subcore_axis_name="s")


def _sc_degree(dst_p, acc_rows):
    ep = dst_p.shape[0]
    per_w = ep // (NC * NS)
    k_ch = per_w // CH
    rows_per_s = acc_rows // NS

    @functools.partial(
        pl.kernel,
        out_type=jax.ShapeDtypeStruct((NC, acc_rows, L), jnp.float32),
        mesh=_sc_mesh(),
        scratch_types=[
            pltpu.VMEM((CH,), jnp.int32),
            pltpu.VMEM((CH, L), jnp.float32),
            pltpu.VMEM_SHARED((acc_rows, L), jnp.float32),
        ],
    )
    def k(dst_hbm, out_hbm, idx_v, ones_v, acc_sh):
        cid = lax.axis_index("c")
        sid = lax.axis_index("s")
        zero = jnp.zeros((L,), jnp.float32)
        one = jnp.ones((L,), jnp.float32)

        @pl.loop(0, CH)
        def _(r):
            ones_v[r, pl.ds(0, L)] = zero

        @pl.loop(0, rows_per_s // CH)
        def _(j):
            pltpu.sync_copy(ones_v,
                            acc_sh.at[pl.ds(sid * rows_per_s + j * CH, CH)])

        @pl.loop(0, CH)
        def _(r):
            ones_v[r, pl.ds(0, L)] = one

        plsc.subcore_barrier()

        base0 = cid * (per_w * NS) + sid * per_w

        @pl.loop(0, k_ch)
        def _(kk):
            pltpu.sync_copy(dst_hbm.at[pl.ds(base0 + kk * CH, CH)], idx_v)
            pltpu.sync_copy(ones_v, acc_sh.at[idx_v], add=True)

        plsc.subcore_barrier()

        @pl.loop(0, rows_per_s // CH)
        def _(j):
            r0 = sid * rows_per_s + j * CH
            pltpu.sync_copy(acc_sh.at[pl.ds(r0, CH)],
                            out_hbm.at[cid, pl.ds(r0, CH)])

    return k(dst_p)


def _sc_aggregate(g, src_p, dst_p, acc_rows):
    ep = src_p.shape[0]
    per_w = ep // (NC * NS)
    k_ch = per_w // CH
    d = g.shape[1]
    rows_per_s = acc_rows // NS

    @functools.partial(
        pl.kernel,
        out_type=jax.ShapeDtypeStruct((NC, acc_rows, d), jnp.float32),
        mesh=_sc_mesh(),
        scratch_types=[
            pltpu.VMEM((CH,), jnp.int32),
            pltpu.VMEM((CH,), jnp.int32),
            pltpu.VMEM((CH,), jnp.int32),
            pltpu.VMEM((CH, d), jnp.float32),
            pltpu.VMEM((CH, d), jnp.float32),
            pltpu.VMEM_SHARED((acc_rows, d), jnp.float32),
            pltpu.SemaphoreType.DMA,
            pltpu.SemaphoreType.DMA,
        ],
    )
    def k(g_hbm, src_hbm, dst_hbm, out_hbm,
          src_va, src_vb, dst_v, rows_a, rows_b, acc_sh, sem_a, sem_b):
        cid = lax.axis_index("c")
        sid = lax.axis_index("s")
        zero = jnp.zeros((L,), jnp.float32)

        @pl.loop(0, CH)
        def _(r):
            @pl.loop(0, d // L)
            def _(cc):
                rows_a[r, pl.ds(cc * L, L)] = zero

        @pl.loop(0, rows_per_s // CH)
        def _(j):
            pltpu.sync_copy(rows_a,
                            acc_sh.at[pl.ds(sid * rows_per_s + j * CH, CH)])

        plsc.subcore_barrier()

        base0 = cid * (per_w * NS) + sid * per_w

        pltpu.sync_copy(src_hbm.at[pl.ds(base0, CH)], src_va)
        pltpu.make_async_copy(g_hbm.at[src_va], rows_a, sem_a).start()

        @pl.loop(0, k_ch // 2)
        def _(kk):
            e0 = base0 + kk * (2 * CH)
            pltpu.sync_copy(src_hbm.at[pl.ds(e0 + CH, CH)], src_vb)
            pltpu.make_async_copy(g_hbm.at[src_vb], rows_b, sem_b).start()
            pltpu.sync_copy(dst_hbm.at[pl.ds(e0, CH)], dst_v)
            pltpu.make_async_copy(g_hbm.at[src_va], rows_a, sem_a).wait()
            pltpu.sync_copy(rows_a, acc_sh.at[dst_v], add=True)

            @pl.when(kk < k_ch // 2 - 1)
            def _():
                pltpu.sync_copy(src_hbm.at[pl.ds(e0 + 2 * CH, CH)], src_va)
                pltpu.make_async_copy(g_hbm.at[src_va], rows_a, sem_a).start()

            pltpu.sync_copy(dst_hbm.at[pl.ds(e0 + CH, CH)], dst_v)
            pltpu.make_async_copy(g_hbm.at[src_vb], rows_b, sem_b).wait()
            pltpu.sync_copy(rows_b, acc_sh.at[dst_v], add=True)

        plsc.subcore_barrier()

        @pl.loop(0, rows_per_s // CH)
        def _(j):
            r0 = sid * rows_per_s + j * CH
            pltpu.sync_copy(acc_sh.at[pl.ds(r0, CH)],
                            out_hbm.at[cid, pl.ds(r0, CH)])

    return k(g, src_p, dst_p)


def _tc_matmul(x, w):
    n, d_in = x.shape
    d_out = w.shape[1]
    blk = 2000

    def body(x_ref, w_ref, o_ref):
        o_ref[...] = jnp.dot(x_ref[...], w_ref[...],
                             preferred_element_type=jnp.float32)

    return pl.pallas_call(
        body,
        grid=(n // blk,),
        in_specs=[pl.BlockSpec((blk, d_in), lambda i: (i, 0)),
                  pl.BlockSpec((d_in, d_out), lambda i: (0, 0))],
        out_specs=pl.BlockSpec((blk, d_out), lambda i: (i, 0)),
        out_shape=jax.ShapeDtypeStruct((n, d_out), jnp.float32),
    )(x, w)


def _tc_scale(h, deg_parts):
    n, d = h.shape
    blk = 2000

    def body(h_ref, p0_ref, p1_ref, g_ref):
        deg = 1.0 + p0_ref[0, :, 0] + p1_ref[0, :, 0]
        g_ref[...] = h_ref[...] * lax.rsqrt(deg)[:, None]

    return pl.pallas_call(
        body,
        grid=(n // blk,),
        in_specs=[pl.BlockSpec((blk, d), lambda i: (i, 0)),
                  pl.BlockSpec((1, blk, L), lambda i: (0, i, 0)),
                  pl.BlockSpec((1, blk, L), lambda i: (1, i, 0))],
        out_specs=pl.BlockSpec((blk, d), lambda i: (i, 0)),
        out_shape=jax.ShapeDtypeStruct((n, d), jnp.float32),
    )(h, deg_parts, deg_parts)


def _tc_final(agg_parts, deg_parts, h, b2):
    n, d = h.shape
    blk = 2000

    def body(a0_ref, a1_ref, p0_ref, p1_ref, h_ref, b_ref, y_ref):
        deg = 1.0 + p0_ref[0, :, 0] + p1_ref[0, :, 0]
        dinv = lax.rsqrt(deg)
        aggr = a0_ref[0] + a1_ref[0]
        s = (aggr * dinv[:, None] + h_ref[...] * (1.0 / deg)[:, None]
             + b_ref[...])
        s = s - jnp.max(s, axis=1, keepdims=True)
        e = jnp.exp(s)
        y_ref[...] = e / jnp.sum(e, axis=1, keepdims=True)

    return pl.pallas_call(
        body,
        grid=(n // blk,),
        in_specs=[pl.BlockSpec((1, blk, d), lambda i: (0, i, 0)),
                  pl.BlockSpec((1, blk, d), lambda i: (1, i, 0)),
                  pl.BlockSpec((1, blk, L), lambda i: (0, i, 0)),
                  pl.BlockSpec((1, blk, L), lambda i: (1, i, 0)),
                  pl.BlockSpec((blk, d), lambda i: (i, 0)),
                  pl.BlockSpec((1, d), lambda i: (0, 0))],
        out_specs=pl.BlockSpec((blk, d), lambda i: (i, 0)),
        out_shape=jax.ShapeDtypeStruct((n, d), jnp.float32),
    )(agg_parts, agg_parts, deg_parts, deg_parts, h, b2)


def kernel(x, edge_index, W, b):
    n = x.shape[0]
    e = edge_index.shape[1]
    src = edge_index[0].astype(jnp.int32)
    dst = edge_index[1].astype(jnp.int32)

    step = 2 * NC * NS * CH
    ep = ((e + step - 1) // step) * step
    pad = ep - e
    src_p = jnp.concatenate([src, jnp.zeros((pad,), jnp.int32)])
    dst_p = jnp.concatenate([dst, jnp.full((pad,), n, jnp.int32)])

    acc_rows = ((n + 1 + NS * CH - 1) // (NS * CH)) * (NS * CH)

    deg_parts = _sc_degree(dst_p, acc_rows)
    h = _tc_matmul(x, W)
    g = _tc_scale(h, deg_parts)
    agg_parts = _sc_aggregate(g, src_p, dst_p, acc_rows)
    return _tc_final(agg_parts, deg_parts, h, b.reshape(1, -1))


# speedup vs baseline: 14.0061x; 1.0476x over previous
import functools

import jax
import jax.numpy as jnp
from jax import lax
from jax.experimental import pallas as pl
from jax.experimental.pallas import tpu as pltpu
from jax.experimental.pallas import tpu_sc as plsc

NC = 2
NS = 16
L = 16
CH = 128


def _sc_mesh():
    return plsc.VectorSubcoreMesh(core_axis_name="c", subcore_axis_name="s")


def _sc_degree(dst_p2, acc_rows):
    n_rows = dst_p2.shape[0]
    k_ch = n_rows // (NC * NS)
    rows_per_s = acc_rows // NS

    @functools.partial(
        pl.kernel,
        out_type=jax.ShapeDtypeStruct((NC, acc_rows, L), jnp.float32),
        mesh=_sc_mesh(),
        scratch_types=[
            pltpu.VMEM((k_ch, CH), jnp.int32),
            pltpu.VMEM((CH, L), jnp.float32),
            pltpu.VMEM_SHARED((acc_rows, L), jnp.float32),
        ],
    )
    def k(dst_hbm, out_hbm, idx_v, ones_v, acc_sh):
        cid = lax.axis_index("c")
        sid = lax.axis_index("s")
        zero = jnp.zeros((L,), jnp.float32)
        one = jnp.ones((L,), jnp.float32)

        @pl.loop(0, CH)
        def _(r):
            ones_v[r, pl.ds(0, L)] = zero

        @pl.loop(0, rows_per_s // CH)
        def _(j):
            pltpu.sync_copy(ones_v,
                            acc_sh.at[pl.ds(sid * rows_per_s + j * CH, CH)])

        @pl.loop(0, CH)
        def _(r):
            ones_v[r, pl.ds(0, L)] = one

        wid = cid * NS + sid
        pltpu.sync_copy(dst_hbm.at[pl.ds(wid * k_ch, k_ch)], idx_v)

        plsc.subcore_barrier()

        @pl.loop(0, k_ch)
        def _(kk):
            pltpu.sync_copy(ones_v, acc_sh.at[idx_v.at[kk]], add=True)

        plsc.subcore_barrier()

        @pl.loop(0, rows_per_s // CH)
        def _(j):
            r0 = sid * rows_per_s + j * CH
            pltpu.sync_copy(acc_sh.at[pl.ds(r0, CH)],
                            out_hbm.at[cid, pl.ds(r0, CH)])

    return k(dst_p2)


def _sc_aggregate(g, src_p2, dst_p2, acc_rows):
    n_rows = src_p2.shape[0]
    k_ch = n_rows // (NC * NS)
    d = g.shape[1]
    rows_per_s = acc_rows // NS

    @functools.partial(
        pl.kernel,
        out_type=jax.ShapeDtypeStruct((NC, acc_rows, d), jnp.float32),
        mesh=_sc_mesh(),
        scratch_types=[
            pltpu.VMEM((CH,), jnp.int32),
            pltpu.VMEM((CH,), jnp.int32),
            pltpu.VMEM((k_ch, CH), jnp.int32),
            pltpu.VMEM((CH, d), jnp.float32),
            pltpu.VMEM((CH, d), jnp.float32),
            pltpu.VMEM_SHARED((acc_rows, d), jnp.float32),
            pltpu.SemaphoreType.DMA,
            pltpu.SemaphoreType.DMA,
        ],
    )
    def k(g_hbm, src_hbm, dst_hbm, out_hbm,
          src_a, src_b, dst_v, rows_a, rows_b, acc_sh, sem_a, sem_b):
        cid = lax.axis_index("c")
        sid = lax.axis_index("s")
        zero = jnp.zeros((L,), jnp.float32)

        @pl.loop(0, CH)
        def _(r):
            @pl.loop(0, d // L)
            def _(cc):
                rows_a[r, pl.ds(cc * L, L)] = zero

        @pl.loop(0, rows_per_s // CH)
        def _(j):
            pltpu.sync_copy(rows_a,
                            acc_sh.at[pl.ds(sid * rows_per_s + j * CH, CH)])

        wid = cid * NS + sid
        pltpu.sync_copy(dst_hbm.at[pl.ds(wid * k_ch, k_ch)], dst_v)

        plsc.subcore_barrier()

        row0 = wid * k_ch

        pltpu.sync_copy(src_hbm.at[row0], src_a)
        pltpu.make_async_copy(g_hbm.at[src_a], rows_a, sem_a).start()

        @pl.loop(0, k_ch // 2)
        def _(kk):
            k0 = 2 * kk
            pltpu.sync_copy(src_hbm.at[row0 + k0 + 1], src_b)
            pltpu.make_async_copy(g_hbm.at[src_b], rows_b, sem_b).start()
            pltpu.make_async_copy(g_hbm.at[src_a], rows_a, sem_a).wait()
            pltpu.sync_copy(rows_a, acc_sh.at[dst_v.at[k0]], add=True)

            @pl.when(kk < k_ch // 2 - 1)
            def _():
                pltpu.sync_copy(src_hbm.at[row0 + k0 + 2], src_a)
                pltpu.make_async_copy(g_hbm.at[src_a], rows_a, sem_a).start()

            pltpu.make_async_copy(g_hbm.at[src_b], rows_b, sem_b).wait()
            pltpu.sync_copy(rows_b, acc_sh.at[dst_v.at[k0 + 1]], add=True)

        plsc.subcore_barrier()

        @pl.loop(0, rows_per_s // CH)
        def _(j):
            r0 = sid * rows_per_s + j * CH
            pltpu.sync_copy(acc_sh.at[pl.ds(r0, CH)],
                            out_hbm.at[cid, pl.ds(r0, CH)])

    return k(g, src_p2, dst_p2)


def _tc_matmul(x, w):
    n, d_in = x.shape
    d_out = w.shape[1]
    blk = 2000

    def body(x_ref, w_ref, o_ref):
        o_ref[...] = jnp.dot(x_ref[...], w_ref[...],
                             preferred_element_type=jnp.float32)

    return pl.pallas_call(
        body,
        grid=(n // blk,),
        in_specs=[pl.BlockSpec((blk, d_in), lambda i: (i, 0)),
                  pl.BlockSpec((d_in, d_out), lambda i: (0, 0))],
        out_specs=pl.BlockSpec((blk, d_out), lambda i: (i, 0)),
        out_shape=jax.ShapeDtypeStruct((n, d_out), jnp.float32),
    )(x, w)


def _tc_scale(h, deg_parts):
    n, d = h.shape
    blk = 2000

    def body(h_ref, p0_ref, p1_ref, g_ref):
        deg = 1.0 + p0_ref[0, :, 0] + p1_ref[0, :, 0]
        g_ref[...] = h_ref[...] * lax.rsqrt(deg)[:, None]

    return pl.pallas_call(
        body,
        grid=(n // blk,),
        in_specs=[pl.BlockSpec((blk, d), lambda i: (i, 0)),
                  pl.BlockSpec((1, blk, L), lambda i: (0, i, 0)),
                  pl.BlockSpec((1, blk, L), lambda i: (1, i, 0))],
        out_specs=pl.BlockSpec((blk, d), lambda i: (i, 0)),
        out_shape=jax.ShapeDtypeStruct((n, d), jnp.float32),
    )(h, deg_parts, deg_parts)


def _tc_final(agg_parts, deg_parts, h, b2):
    n, d = h.shape
    blk = 2000

    def body(a0_ref, a1_ref, p0_ref, p1_ref, h_ref, b_ref, y_ref):
        deg = 1.0 + p0_ref[0, :, 0] + p1_ref[0, :, 0]
        dinv = lax.rsqrt(deg)
        aggr = a0_ref[0] + a1_ref[0]
        s = (aggr * dinv[:, None] + h_ref[...] * (1.0 / deg)[:, None]
             + b_ref[...])
        s = s - jnp.max(s, axis=1, keepdims=True)
        e = jnp.exp(s)
        y_ref[...] = e / jnp.sum(e, axis=1, keepdims=True)

    return pl.pallas_call(
        body,
        grid=(n // blk,),
        in_specs=[pl.BlockSpec((1, blk, d), lambda i: (0, i, 0)),
                  pl.BlockSpec((1, blk, d), lambda i: (1, i, 0)),
                  pl.BlockSpec((1, blk, L), lambda i: (0, i, 0)),
                  pl.BlockSpec((1, blk, L), lambda i: (1, i, 0)),
                  pl.BlockSpec((blk, d), lambda i: (i, 0)),
                  pl.BlockSpec((1, d), lambda i: (0, 0))],
        out_specs=pl.BlockSpec((blk, d), lambda i: (i, 0)),
        out_shape=jax.ShapeDtypeStruct((n, d), jnp.float32),
    )(agg_parts, agg_parts, deg_parts, deg_parts, h, b2)


def kernel(x, edge_index, W, b):
    n = x.shape[0]
    e = edge_index.shape[1]
    src = edge_index[0].astype(jnp.int32)
    dst = edge_index[1].astype(jnp.int32)

    step = 2 * NC * NS * CH
    ep = ((e + step - 1) // step) * step
    pad = ep - e
    src_p2 = jnp.concatenate([src, jnp.zeros((pad,), jnp.int32)]).reshape(
        ep // CH, CH)
    dst_p2 = jnp.concatenate([dst, jnp.full((pad,), n, jnp.int32)]).reshape(
        ep // CH, CH)

    acc_rows = ((n + 1 + NS * CH - 1) // (NS * CH)) * (NS * CH)

    deg_parts = _sc_degree(dst_p2, acc_rows)
    h = _tc_matmul(x, W)
    g = _tc_scale(h, deg_parts)
    agg_parts = _sc_aggregate(g, src_p2, dst_p2, acc_rows)
    return _tc_final(agg_parts, deg_parts, h, b.reshape(1, -1))
